# Initial kernel scaffold; baseline (speedup 1.0000x reference)
import jax
import jax.numpy as jnp
from jax.experimental import pallas as pl


def kernel(triple_idx_i, triple_idx_j, triple_idx_k, idx_i, idx_j, z, positions):
    raise NotImplementedError("write your pallas kernel here")


# speedup vs baseline: 7.7335x; 7.7335x over previous
import jax
import jax.numpy as jnp
import numpy as np
from jax import lax
from jax.experimental import pallas as pl
from jax.experimental.pallas import tpu as pltpu
from jax.experimental.pallas import tpu_sc as plsc

N_NODES = 10000
N_TRIPLES = 640000
CFG_W = 22
OUT_W = 3 * CFG_W + 3
NZ = 100
IW = 24
JW = 40

_ORB = '1s 2s 2p 3s 3p 4s 3d 4p 5s 4d 5p 6s 4f 5d 6p 7s 5f 6d 7p 6f 7d 7f'.split()
_CAP = dict(s=2, p=6, d=10, f=14)


def _cfg_row(a):
    cnt, last, row = 0, -1, []
    for o in _ORB:
        if cnt < a:
            c = _CAP[o[-1]]
            row.append(c)
            cnt += c
            last += 1
        else:
            row.append(0)
    if cnt > a:
        row[last] -= cnt - a
    return row


_TAB_NP = np.array([_cfg_row(a) for a in range(NZ)], dtype=np.float32)

_ITAB_NP = np.zeros((NZ * NZ, IW), dtype=np.float32)
_ITAB_NP[:, 0:CFG_W] = np.repeat(_TAB_NP, NZ, axis=0)
_ITAB_NP[:, CFG_W:IW] = np.tile(_TAB_NP[:, 0:2], (NZ, 1))

_JTAB_NP = np.zeros((NZ * NZ, JW), dtype=np.float32)
_JTAB_NP[:, 0:CFG_W - 2] = np.repeat(_TAB_NP[:, 2:CFG_W], NZ, axis=0)
_JTAB_NP[:, CFG_W - 2:JW] = np.tile(_TAB_NP[:, 0:CFG_W - 2], (NZ, 1))

_T20_NP = np.ascontiguousarray(_TAB_NP[:, 20])
_T21_NP = np.ascontiguousarray(_TAB_NP[:, 21])

NW = 32
CHUNK = 80
PER_W = N_TRIPLES // NW
N_CH = PER_W // CHUNK
VEC_IT = CHUNK // 16


def _sc_body(ti_hbm, tj_hbm, tk_hbm, idxj_hbm, z_hbm, px_hbm, py_hbm,
             itab_hbm, jtab_hbm, t20_hbm, t21_hbm,
             out_hbm,
             itab_sp, jtab_sp, px_v, py_v, t20_v, t21_v,
             ti_v, tj_v, tk_v, jn_v, kn_v, zi_v, zj_v, zk_v, pidx_v, qidx_v,
             out_v, ti_b, tj_b):
    cid = lax.axis_index("c")
    sid = lax.axis_index("s")
    wid = sid * 2 + cid
    base = wid * PER_W

    @pl.when(sid == 0)
    def _():
        pltpu.sync_copy(itab_hbm, itab_sp)
        pltpu.sync_copy(jtab_hbm, jtab_sp)

    pltpu.sync_copy(px_hbm, px_v)
    pltpu.sync_copy(py_hbm, py_v)
    pltpu.sync_copy(t20_hbm, t20_v)
    pltpu.sync_copy(t21_hbm, t21_v)
    plsc.subcore_barrier()

    c64 = jnp.full((16,), 64, jnp.int32)
    c65 = jnp.full((16,), 65, jnp.int32)
    c66 = jnp.full((16,), 66, jnp.int32)
    c67 = jnp.full((16,), 67, jnp.int32)
    c68 = jnp.full((16,), 68, jnp.int32)

    def chunk_body(ci, _):
        g0 = base + ci * CHUNK
        pltpu.sync_copy(ti_hbm.at[pl.ds(g0, CHUNK)], ti_v)
        pltpu.sync_copy(tj_hbm.at[pl.ds(g0, CHUNK)], tj_v)
        pltpu.sync_copy(tk_hbm.at[pl.ds(g0, CHUNK)], tk_v)
        pltpu.sync_copy(idxj_hbm.at[tj_v], jn_v)
        pltpu.sync_copy(idxj_hbm.at[tk_v], kn_v)
        pltpu.sync_copy(z_hbm.at[ti_v], zi_v)
        pltpu.sync_copy(z_hbm.at[jn_v], zj_v)
        pltpu.sync_copy(z_hbm.at[kn_v], zk_v)

        def pidx_body(vi, _):
            s = pl.ds(vi * 16, 16)
            zj = zj_v[s]
            pidx_v[s] = zj * 100 + zk_v[s]
            qidx_v[s] = zi_v[s] * 100 + zj
            return _

        lax.fori_loop(0, VEC_IT, pidx_body, 0, unroll=4)

        pltpu.sync_copy(itab_sp.at[qidx_v], ti_b)
        pltpu.sync_copy(jtab_sp.at[pidx_v], tj_b)

        def vec_body(vi, _):
            row16 = lax.iota(jnp.int32, 16) + vi * 16
            ti16 = ti_v[pl.ds(vi * 16, 16)]
            jn16 = jn_v[pl.ds(vi * 16, 16)]
            kn16 = kn_v[pl.ds(vi * 16, 16)]
            zk16 = zk_v[pl.ds(vi * 16, 16)]
            pxi = plsc.load_gather(px_v, [ti16])
            pxj = plsc.load_gather(px_v, [jn16])
            pxk = plsc.load_gather(px_v, [kn16])
            pyi = plsc.load_gather(py_v, [ti16])
            pyj = plsc.load_gather(py_v, [jn16])
            pyk = plsc.load_gather(py_v, [kn16])
            t20 = plsc.load_gather(t20_v, [zk16])
            t21 = plsc.load_gather(t21_v, [zk16])
            sx = ((pxj - pxi) + (pxk - pxi)) * 0.5
            y2 = pyj - pyi
            y3 = pyk - pyi
            plsc.store_scatter(out_v, [row16, c64], t20)
            plsc.store_scatter(out_v, [row16, c65], t21)
            plsc.store_scatter(out_v, [row16, c66], sx)
            plsc.store_scatter(out_v, [row16, c67], y2)
            plsc.store_scatter(out_v, [row16, c68], y3)
            return _

        lax.fori_loop(0, VEC_IT, vec_body, 0, unroll=2)
        pltpu.sync_copy(out_v, out_hbm.at[pl.ds(g0, CHUNK)])
        pltpu.sync_copy(ti_b, out_hbm.at[pl.ds(g0, CHUNK), pl.ds(0, IW)])
        pltpu.sync_copy(tj_b, out_hbm.at[pl.ds(g0, CHUNK), pl.ds(IW, JW)])
        return _

    lax.fori_loop(0, N_CH, chunk_body, 0)


@jax.jit
def kernel(triple_idx_i, triple_idx_j, triple_idx_k, idx_i, idx_j, z, positions):
    del idx_i
    itab = jnp.asarray(_ITAB_NP)
    jtab = jnp.asarray(_JTAB_NP)
    t20 = jnp.asarray(_T20_NP)
    t21 = jnp.asarray(_T21_NP)
    px = positions[:, 0]
    py = positions[:, 1]
    mesh = plsc.VectorSubcoreMesh(core_axis_name="c", subcore_axis_name="s")
    f = pl.kernel(
        _sc_body,
        out_type=jax.ShapeDtypeStruct((N_TRIPLES, OUT_W), jnp.float32),
        mesh=mesh,
        compiler_params=pltpu.CompilerParams(
            needs_layout_passes=False, use_tc_tiling_on_sc=False),
        scratch_types=[
            pltpu.VMEM_SHARED((NZ * NZ, IW), jnp.float32),
            pltpu.VMEM_SHARED((NZ * NZ, JW), jnp.float32),
            pltpu.VMEM((N_NODES,), jnp.float32),
            pltpu.VMEM((N_NODES,), jnp.float32),
            pltpu.VMEM((NZ,), jnp.float32),
            pltpu.VMEM((NZ,), jnp.float32),
            pltpu.VMEM((CHUNK,), jnp.int32),
            pltpu.VMEM((CHUNK,), jnp.int32),
            pltpu.VMEM((CHUNK,), jnp.int32),
            pltpu.VMEM((CHUNK,), jnp.int32),
            pltpu.VMEM((CHUNK,), jnp.int32),
            pltpu.VMEM((CHUNK,), jnp.int32),
            pltpu.VMEM((CHUNK,), jnp.int32),
            pltpu.VMEM((CHUNK,), jnp.int32),
            pltpu.VMEM((CHUNK,), jnp.int32),
            pltpu.VMEM((CHUNK,), jnp.int32),
            pltpu.VMEM((CHUNK, OUT_W), jnp.float32),
            pltpu.VMEM((CHUNK, IW), jnp.float32),
            pltpu.VMEM((CHUNK, JW), jnp.float32),
        ],
    )
    return f(triple_idx_i, triple_idx_j, triple_idx_k, idx_j, z, px, py,
             itab, jtab, t20, t21)


# speedup vs baseline: 13.8165x; 1.7866x over previous
import jax
import jax.numpy as jnp
import numpy as np
from jax import lax
from jax.experimental import pallas as pl
from jax.experimental.pallas import tpu as pltpu
from jax.experimental.pallas import tpu_sc as plsc

N_NODES = 10000
N_TRIPLES = 640000
CFG_W = 22
OUT_W = 3 * CFG_W + 3
NZ = 100
IW = 24
JW = 40

_ORB = '1s 2s 2p 3s 3p 4s 3d 4p 5s 4d 5p 6s 4f 5d 6p 7s 5f 6d 7p 6f 7d 7f'.split()
_CAP = dict(s=2, p=6, d=10, f=14)


def _cfg_row(a):
    cnt, last, row = 0, -1, []
    for o in _ORB:
        if cnt < a:
            c = _CAP[o[-1]]
            row.append(c)
            cnt += c
            last += 1
        else:
            row.append(0)
    if cnt > a:
        row[last] -= cnt - a
    return row


_TAB_NP = np.array([_cfg_row(a) for a in range(NZ)], dtype=np.float32)

_ITAB_NP = np.zeros((NZ * NZ, IW), dtype=np.float32)
_ITAB_NP[:, 0:CFG_W] = np.repeat(_TAB_NP, NZ, axis=0)
_ITAB_NP[:, CFG_W:IW] = np.tile(_TAB_NP[:, 0:2], (NZ, 1))

_JTAB_NP = np.zeros((NZ * NZ, JW), dtype=np.float32)
_JTAB_NP[:, 0:CFG_W - 2] = np.repeat(_TAB_NP[:, 2:CFG_W], NZ, axis=0)
_JTAB_NP[:, CFG_W - 2:JW] = np.tile(_TAB_NP[:, 0:CFG_W - 2], (NZ, 1))

_T20_NP = np.ascontiguousarray(_TAB_NP[:, 20])
_T21_NP = np.ascontiguousarray(_TAB_NP[:, 21])

NW = 32
CHUNK = 400
PER_W = N_TRIPLES // NW
N_CH = PER_W // CHUNK
VEC_IT = CHUNK // 16


def _sc_body(ti_hbm, tj_hbm, tk_hbm, idxj_hbm, z_hbm, px_hbm, py_hbm,
             itab_hbm, jtab_hbm, t20_hbm, t21_hbm,
             out_hbm,
             itab_sp, jtab_sp, px_v, py_v, t20_v, t21_v,
             ti_v, tj_v, tk_v, jn_v, kn_v, zi_v, zj_v, zk_v, pidx_v, qidx_v,
             out_v, ti_b, tj_b):
    cid = lax.axis_index("c")
    sid = lax.axis_index("s")
    wid = sid * 2 + cid
    base = wid * PER_W

    @pl.when(sid == 0)
    def _():
        pltpu.sync_copy(itab_hbm, itab_sp)
        pltpu.sync_copy(jtab_hbm, jtab_sp)

    pltpu.sync_copy(px_hbm, px_v)
    pltpu.sync_copy(py_hbm, py_v)
    pltpu.sync_copy(t20_hbm, t20_v)
    pltpu.sync_copy(t21_hbm, t21_v)
    plsc.subcore_barrier()

    c64 = jnp.full((16,), 64, jnp.int32)
    c65 = jnp.full((16,), 65, jnp.int32)
    c66 = jnp.full((16,), 66, jnp.int32)
    c67 = jnp.full((16,), 67, jnp.int32)
    c68 = jnp.full((16,), 68, jnp.int32)

    def chunk_body(ci, _):
        g0 = base + ci * CHUNK
        pltpu.sync_copy(ti_hbm.at[pl.ds(g0, CHUNK)], ti_v)
        pltpu.sync_copy(tj_hbm.at[pl.ds(g0, CHUNK)], tj_v)
        pltpu.sync_copy(tk_hbm.at[pl.ds(g0, CHUNK)], tk_v)
        pltpu.sync_copy(idxj_hbm.at[tj_v], jn_v)
        pltpu.sync_copy(idxj_hbm.at[tk_v], kn_v)
        pltpu.sync_copy(z_hbm.at[ti_v], zi_v)
        pltpu.sync_copy(z_hbm.at[jn_v], zj_v)
        pltpu.sync_copy(z_hbm.at[kn_v], zk_v)

        def pidx_body(vi, _):
            s = pl.ds(vi * 16, 16)
            zj = zj_v[s]
            pidx_v[s] = zj * 100 + zk_v[s]
            qidx_v[s] = zi_v[s] * 100 + zj
            return _

        lax.fori_loop(0, VEC_IT, pidx_body, 0, unroll=4)

        pltpu.sync_copy(itab_sp.at[qidx_v], ti_b)
        pltpu.sync_copy(jtab_sp.at[pidx_v], tj_b)

        def vec_body(vi, _):
            row16 = lax.iota(jnp.int32, 16) + vi * 16
            ti16 = ti_v[pl.ds(vi * 16, 16)]
            jn16 = jn_v[pl.ds(vi * 16, 16)]
            kn16 = kn_v[pl.ds(vi * 16, 16)]
            zk16 = zk_v[pl.ds(vi * 16, 16)]
            pxi = plsc.load_gather(px_v, [ti16])
            pxj = plsc.load_gather(px_v, [jn16])
            pxk = plsc.load_gather(px_v, [kn16])
            pyi = plsc.load_gather(py_v, [ti16])
            pyj = plsc.load_gather(py_v, [jn16])
            pyk = plsc.load_gather(py_v, [kn16])
            t20 = plsc.load_gather(t20_v, [zk16])
            t21 = plsc.load_gather(t21_v, [zk16])
            sx = ((pxj - pxi) + (pxk - pxi)) * 0.5
            y2 = pyj - pyi
            y3 = pyk - pyi
            plsc.store_scatter(out_v, [row16, c64], t20)
            plsc.store_scatter(out_v, [row16, c65], t21)
            plsc.store_scatter(out_v, [row16, c66], sx)
            plsc.store_scatter(out_v, [row16, c67], y2)
            plsc.store_scatter(out_v, [row16, c68], y3)
            return _

        lax.fori_loop(0, VEC_IT, vec_body, 0, unroll=2)
        pltpu.sync_copy(out_v, out_hbm.at[pl.ds(g0, CHUNK)])
        pltpu.sync_copy(ti_b, out_hbm.at[pl.ds(g0, CHUNK), pl.ds(0, IW)])
        pltpu.sync_copy(tj_b, out_hbm.at[pl.ds(g0, CHUNK), pl.ds(IW, JW)])
        return _

    lax.fori_loop(0, N_CH, chunk_body, 0)


@jax.jit
def kernel(triple_idx_i, triple_idx_j, triple_idx_k, idx_i, idx_j, z, positions):
    del idx_i
    itab = jnp.asarray(_ITAB_NP)
    jtab = jnp.asarray(_JTAB_NP)
    t20 = jnp.asarray(_T20_NP)
    t21 = jnp.asarray(_T21_NP)
    px = positions[:, 0]
    py = positions[:, 1]
    mesh = plsc.VectorSubcoreMesh(core_axis_name="c", subcore_axis_name="s")
    f = pl.kernel(
        _sc_body,
        out_type=jax.ShapeDtypeStruct((N_TRIPLES, OUT_W), jnp.float32),
        mesh=mesh,
        compiler_params=pltpu.CompilerParams(
            needs_layout_passes=False, use_tc_tiling_on_sc=False),
        scratch_types=[
            pltpu.VMEM_SHARED((NZ * NZ, IW), jnp.float32),
            pltpu.VMEM_SHARED((NZ * NZ, JW), jnp.float32),
            pltpu.VMEM((N_NODES,), jnp.float32),
            pltpu.VMEM((N_NODES,), jnp.float32),
            pltpu.VMEM((NZ,), jnp.float32),
            pltpu.VMEM((NZ,), jnp.float32),
            pltpu.VMEM((CHUNK,), jnp.int32),
            pltpu.VMEM((CHUNK,), jnp.int32),
            pltpu.VMEM((CHUNK,), jnp.int32),
            pltpu.VMEM((CHUNK,), jnp.int32),
            pltpu.VMEM((CHUNK,), jnp.int32),
            pltpu.VMEM((CHUNK,), jnp.int32),
            pltpu.VMEM((CHUNK,), jnp.int32),
            pltpu.VMEM((CHUNK,), jnp.int32),
            pltpu.VMEM((CHUNK,), jnp.int32),
            pltpu.VMEM((CHUNK,), jnp.int32),
            pltpu.VMEM((CHUNK, OUT_W), jnp.float32),
            pltpu.VMEM((CHUNK, IW), jnp.float32),
            pltpu.VMEM((CHUNK, JW), jnp.float32),
        ],
    )
    return f(triple_idx_i, triple_idx_j, triple_idx_k, idx_j, z, px, py,
             itab, jtab, t20, t21)


# speedup vs baseline: 15.9991x; 1.1580x over previous
import jax
import jax.numpy as jnp
import numpy as np
from jax import lax
from jax.experimental import pallas as pl
from jax.experimental.pallas import tpu as pltpu
from jax.experimental.pallas import tpu_sc as plsc

N_NODES = 10000
N_TRIPLES = 640000
CFG_W = 22
OUT_W = 3 * CFG_W + 3
NZ = 100
IW = 24
JW = 40

_ORB = '1s 2s 2p 3s 3p 4s 3d 4p 5s 4d 5p 6s 4f 5d 6p 7s 5f 6d 7p 6f 7d 7f'.split()
_CAP = dict(s=2, p=6, d=10, f=14)


def _cfg_row(a):
    cnt, last, row = 0, -1, []
    for o in _ORB:
        if cnt < a:
            c = _CAP[o[-1]]
            row.append(c)
            cnt += c
            last += 1
        else:
            row.append(0)
    if cnt > a:
        row[last] -= cnt - a
    return row


_TAB_NP = np.array([_cfg_row(a) for a in range(NZ)], dtype=np.float32)

_ITAB_NP = np.zeros((NZ * NZ, IW), dtype=np.float32)
_ITAB_NP[:, 0:CFG_W] = np.repeat(_TAB_NP, NZ, axis=0)
_ITAB_NP[:, CFG_W:IW] = np.tile(_TAB_NP[:, 0:2], (NZ, 1))

_JTAB_NP = np.zeros((NZ * NZ, JW), dtype=np.float32)
_JTAB_NP[:, 0:CFG_W - 2] = np.repeat(_TAB_NP[:, 2:CFG_W], NZ, axis=0)
_JTAB_NP[:, CFG_W - 2:JW] = np.tile(_TAB_NP[:, 0:CFG_W - 2], (NZ, 1))

_T20_NP = np.ascontiguousarray(_TAB_NP[:, 20])
_T21_NP = np.ascontiguousarray(_TAB_NP[:, 21])

NW = 32
CHUNK = 400
PER_W = N_TRIPLES // NW
N_CH = PER_W // CHUNK
VEC_IT = CHUNK // 16


def _sc_body(ti_hbm, tj_hbm, tk_hbm, idxj_hbm, z_hbm, px_hbm, py_hbm,
             itab_hbm, jtab_hbm, t20_hbm, t21_hbm,
             out_hbm,
             itab_sp, jtab_sp, px_v, py_v, t20_v, t21_v,
             ti_v, tj_v, tk_v, jn_v, kn_v, zi_v, zj_v, zk_v, pidx_v, qidx_v,
             out_v, ti_b, tj_b, sem):
    cid = lax.axis_index("c")
    sid = lax.axis_index("s")
    wid = sid * 2 + cid
    base = wid * PER_W

    @pl.when(sid == 0)
    def _():
        pltpu.sync_copy(itab_hbm, itab_sp)
        pltpu.sync_copy(jtab_hbm, jtab_sp)

    pltpu.sync_copy(px_hbm, px_v)
    pltpu.sync_copy(py_hbm, py_v)
    pltpu.sync_copy(t20_hbm, t20_v)
    pltpu.sync_copy(t21_hbm, t21_v)
    plsc.subcore_barrier()

    c64 = jnp.full((16,), 64, jnp.int32)
    c65 = jnp.full((16,), 65, jnp.int32)
    c66 = jnp.full((16,), 66, jnp.int32)
    c67 = jnp.full((16,), 67, jnp.int32)
    c68 = jnp.full((16,), 68, jnp.int32)

    def chunk_body(ci, _):
        g0 = base + ci * CHUNK
        a1 = pltpu.async_copy(ti_hbm.at[pl.ds(g0, CHUNK)], ti_v, sem)
        a2 = pltpu.async_copy(tj_hbm.at[pl.ds(g0, CHUNK)], tj_v, sem)
        a3 = pltpu.async_copy(tk_hbm.at[pl.ds(g0, CHUNK)], tk_v, sem)
        a1.wait(); a2.wait(); a3.wait()
        b1 = pltpu.async_copy(idxj_hbm.at[tj_v], jn_v, sem)
        b2 = pltpu.async_copy(idxj_hbm.at[tk_v], kn_v, sem)
        c1 = pltpu.async_copy(z_hbm.at[ti_v], zi_v, sem)
        b1.wait(); b2.wait()
        c2 = pltpu.async_copy(z_hbm.at[jn_v], zj_v, sem)
        c3 = pltpu.async_copy(z_hbm.at[kn_v], zk_v, sem)
        c1.wait(); c2.wait(); c3.wait()

        def pidx_body(vi, _):
            s = pl.ds(vi * 16, 16)
            zj = zj_v[s]
            pidx_v[s] = zj * 100 + zk_v[s]
            qidx_v[s] = zi_v[s] * 100 + zj
            return _

        lax.fori_loop(0, VEC_IT, pidx_body, 0, unroll=4)

        e1 = pltpu.async_copy(itab_sp.at[qidx_v], ti_b, sem)
        e2 = pltpu.async_copy(jtab_sp.at[pidx_v], tj_b, sem)

        def vec_body(vi, _):
            row16 = lax.iota(jnp.int32, 16) + vi * 16
            ti16 = ti_v[pl.ds(vi * 16, 16)]
            jn16 = jn_v[pl.ds(vi * 16, 16)]
            kn16 = kn_v[pl.ds(vi * 16, 16)]
            zk16 = zk_v[pl.ds(vi * 16, 16)]
            pxi = plsc.load_gather(px_v, [ti16])
            pxj = plsc.load_gather(px_v, [jn16])
            pxk = plsc.load_gather(px_v, [kn16])
            pyi = plsc.load_gather(py_v, [ti16])
            pyj = plsc.load_gather(py_v, [jn16])
            pyk = plsc.load_gather(py_v, [kn16])
            t20 = plsc.load_gather(t20_v, [zk16])
            t21 = plsc.load_gather(t21_v, [zk16])
            sx = ((pxj - pxi) + (pxk - pxi)) * 0.5
            y2 = pyj - pyi
            y3 = pyk - pyi
            plsc.store_scatter(out_v, [row16, c64], t20)
            plsc.store_scatter(out_v, [row16, c65], t21)
            plsc.store_scatter(out_v, [row16, c66], sx)
            plsc.store_scatter(out_v, [row16, c67], y2)
            plsc.store_scatter(out_v, [row16, c68], y3)
            return _

        lax.fori_loop(0, VEC_IT, vec_body, 0, unroll=2)
        e1.wait(); e2.wait()
        g1 = pltpu.async_copy(out_v, out_hbm.at[pl.ds(g0, CHUNK)], sem)
        g1.wait()
        g2 = pltpu.async_copy(ti_b, out_hbm.at[pl.ds(g0, CHUNK), pl.ds(0, IW)], sem)
        g3 = pltpu.async_copy(tj_b, out_hbm.at[pl.ds(g0, CHUNK), pl.ds(IW, JW)], sem)
        g2.wait(); g3.wait()
        return _

    lax.fori_loop(0, N_CH, chunk_body, 0)


@jax.jit
def kernel(triple_idx_i, triple_idx_j, triple_idx_k, idx_i, idx_j, z, positions):
    del idx_i
    itab = jnp.asarray(_ITAB_NP)
    jtab = jnp.asarray(_JTAB_NP)
    t20 = jnp.asarray(_T20_NP)
    t21 = jnp.asarray(_T21_NP)
    px = positions[:, 0]
    py = positions[:, 1]
    mesh = plsc.VectorSubcoreMesh(core_axis_name="c", subcore_axis_name="s")
    f = pl.kernel(
        _sc_body,
        out_type=jax.ShapeDtypeStruct((N_TRIPLES, OUT_W), jnp.float32),
        mesh=mesh,
        compiler_params=pltpu.CompilerParams(
            needs_layout_passes=False, use_tc_tiling_on_sc=False),
        scratch_types=[
            pltpu.VMEM_SHARED((NZ * NZ, IW), jnp.float32),
            pltpu.VMEM_SHARED((NZ * NZ, JW), jnp.float32),
            pltpu.VMEM((N_NODES,), jnp.float32),
            pltpu.VMEM((N_NODES,), jnp.float32),
            pltpu.VMEM((NZ,), jnp.float32),
            pltpu.VMEM((NZ,), jnp.float32),
            pltpu.VMEM((CHUNK,), jnp.int32),
            pltpu.VMEM((CHUNK,), jnp.int32),
            pltpu.VMEM((CHUNK,), jnp.int32),
            pltpu.VMEM((CHUNK,), jnp.int32),
            pltpu.VMEM((CHUNK,), jnp.int32),
            pltpu.VMEM((CHUNK,), jnp.int32),
            pltpu.VMEM((CHUNK,), jnp.int32),
            pltpu.VMEM((CHUNK,), jnp.int32),
            pltpu.VMEM((CHUNK,), jnp.int32),
            pltpu.VMEM((CHUNK,), jnp.int32),
            pltpu.VMEM((CHUNK, OUT_W), jnp.float32),
            pltpu.VMEM((CHUNK, IW), jnp.float32),
            pltpu.VMEM((CHUNK, JW), jnp.float32),
            pltpu.SemaphoreType.DMA,
        ],
    )
    return f(triple_idx_i, triple_idx_j, triple_idx_k, idx_j, z, px, py,
             itab, jtab, t20, t21)


# speedup vs baseline: 16.6370x; 1.0399x over previous
import jax
import jax.numpy as jnp
import numpy as np
from jax import lax
from jax.experimental import pallas as pl
from jax.experimental.pallas import tpu as pltpu
from jax.experimental.pallas import tpu_sc as plsc

N_NODES = 10000
N_TRIPLES = 640000
CFG_W = 22
OUT_W = 3 * CFG_W + 3
PAD_W = 72
TAIL_W = 8
NZ = 100
IW = 24
JW = 40

_ORB = '1s 2s 2p 3s 3p 4s 3d 4p 5s 4d 5p 6s 4f 5d 6p 7s 5f 6d 7p 6f 7d 7f'.split()
_CAP = dict(s=2, p=6, d=10, f=14)


def _cfg_row(a):
    cnt, last, row = 0, -1, []
    for o in _ORB:
        if cnt < a:
            c = _CAP[o[-1]]
            row.append(c)
            cnt += c
            last += 1
        else:
            row.append(0)
    if cnt > a:
        row[last] -= cnt - a
    return row


_TAB_NP = np.array([_cfg_row(a) for a in range(NZ)], dtype=np.float32)

_ITAB_NP = np.zeros((NZ * NZ, IW), dtype=np.float32)
_ITAB_NP[:, 0:CFG_W] = np.repeat(_TAB_NP, NZ, axis=0)
_ITAB_NP[:, CFG_W:IW] = np.tile(_TAB_NP[:, 0:2], (NZ, 1))

_JTAB_NP = np.zeros((NZ * NZ, JW), dtype=np.float32)
_JTAB_NP[:, 0:CFG_W - 2] = np.repeat(_TAB_NP[:, 2:CFG_W], NZ, axis=0)
_JTAB_NP[:, CFG_W - 2:JW] = np.tile(_TAB_NP[:, 0:CFG_W - 2], (NZ, 1))

_T20_NP = np.ascontiguousarray(_TAB_NP[:, 20])
_T21_NP = np.ascontiguousarray(_TAB_NP[:, 21])

NW = 32
CHUNK = 400
PER_W = N_TRIPLES // NW
N_CH = PER_W // CHUNK
VEC_IT = CHUNK // 16


def _sc_body(ti_hbm, tj_hbm, tk_hbm, idxj_hbm, z_hbm, px_hbm, py_hbm,
             itab_hbm, jtab_hbm, t20_hbm, t21_hbm,
             out_hbm,
             itab_sp, jtab_sp, px_v, py_v, t20_v, t21_v,
             ti_v, tj_v, tk_v, jn_v, kn_v, zi_v, zj_v, zk_v, pidx_v, qidx_v,
             tail_b, ti_b, tj_b, sem):
    cid = lax.axis_index("c")
    sid = lax.axis_index("s")
    wid = sid * 2 + cid
    base = wid * PER_W

    @pl.when(sid == 0)
    def _():
        pltpu.sync_copy(itab_hbm, itab_sp)
        pltpu.sync_copy(jtab_hbm, jtab_sp)

    pltpu.sync_copy(px_hbm, px_v)
    pltpu.sync_copy(py_hbm, py_v)
    pltpu.sync_copy(t20_hbm, t20_v)
    pltpu.sync_copy(t21_hbm, t21_v)
    plsc.subcore_barrier()

    k0 = jnp.full((16,), 0, jnp.int32)
    k1 = jnp.full((16,), 1, jnp.int32)
    k2 = jnp.full((16,), 2, jnp.int32)
    k3 = jnp.full((16,), 3, jnp.int32)
    k4 = jnp.full((16,), 4, jnp.int32)

    def chunk_body(ci, _):
        g0 = base + ci * CHUNK
        a1 = pltpu.async_copy(ti_hbm.at[pl.ds(g0, CHUNK)], ti_v, sem)
        a2 = pltpu.async_copy(tj_hbm.at[pl.ds(g0, CHUNK)], tj_v, sem)
        a3 = pltpu.async_copy(tk_hbm.at[pl.ds(g0, CHUNK)], tk_v, sem)
        a1.wait(); a2.wait(); a3.wait()
        b1 = pltpu.async_copy(idxj_hbm.at[tj_v], jn_v, sem)
        b2 = pltpu.async_copy(idxj_hbm.at[tk_v], kn_v, sem)
        c1 = pltpu.async_copy(z_hbm.at[ti_v], zi_v, sem)
        b1.wait(); b2.wait()
        c2 = pltpu.async_copy(z_hbm.at[jn_v], zj_v, sem)
        c3 = pltpu.async_copy(z_hbm.at[kn_v], zk_v, sem)
        c1.wait(); c2.wait(); c3.wait()

        def pidx_body(vi, _):
            s = pl.ds(vi * 16, 16)
            zj = zj_v[s]
            pidx_v[s] = zj * 100 + zk_v[s]
            qidx_v[s] = zi_v[s] * 100 + zj
            return _

        lax.fori_loop(0, VEC_IT, pidx_body, 0, unroll=4)

        e1 = pltpu.async_copy(itab_sp.at[qidx_v], ti_b, sem)
        e2 = pltpu.async_copy(jtab_sp.at[pidx_v], tj_b, sem)

        def vec_body(vi, _):
            row16 = lax.iota(jnp.int32, 16) + vi * 16
            ti16 = ti_v[pl.ds(vi * 16, 16)]
            jn16 = jn_v[pl.ds(vi * 16, 16)]
            kn16 = kn_v[pl.ds(vi * 16, 16)]
            zk16 = zk_v[pl.ds(vi * 16, 16)]
            pxi = plsc.load_gather(px_v, [ti16])
            pxj = plsc.load_gather(px_v, [jn16])
            pxk = plsc.load_gather(px_v, [kn16])
            pyi = plsc.load_gather(py_v, [ti16])
            pyj = plsc.load_gather(py_v, [jn16])
            pyk = plsc.load_gather(py_v, [kn16])
            t20 = plsc.load_gather(t20_v, [zk16])
            t21 = plsc.load_gather(t21_v, [zk16])
            sx = ((pxj - pxi) + (pxk - pxi)) * 0.5
            y2 = pyj - pyi
            y3 = pyk - pyi
            plsc.store_scatter(tail_b, [row16, k0], t20)
            plsc.store_scatter(tail_b, [row16, k1], t21)
            plsc.store_scatter(tail_b, [row16, k2], sx)
            plsc.store_scatter(tail_b, [row16, k3], y2)
            plsc.store_scatter(tail_b, [row16, k4], y3)
            return _

        lax.fori_loop(0, VEC_IT, vec_body, 0, unroll=2)
        e1.wait(); e2.wait()
        h0 = pltpu.async_copy(
            tail_b, out_hbm.at[pl.ds(g0, CHUNK), pl.ds(JW + IW, TAIL_W)], sem)
        h1 = pltpu.async_copy(
            ti_b, out_hbm.at[pl.ds(g0, CHUNK), pl.ds(0, IW)], sem)
        h2 = pltpu.async_copy(
            tj_b, out_hbm.at[pl.ds(g0, CHUNK), pl.ds(IW, JW)], sem)
        h0.wait(); h1.wait(); h2.wait()
        return _

    lax.fori_loop(0, N_CH, chunk_body, 0)


@jax.jit
def kernel(triple_idx_i, triple_idx_j, triple_idx_k, idx_i, idx_j, z, positions):
    del idx_i
    itab = jnp.asarray(_ITAB_NP)
    jtab = jnp.asarray(_JTAB_NP)
    t20 = jnp.asarray(_T20_NP)
    t21 = jnp.asarray(_T21_NP)
    px = positions[:, 0]
    py = positions[:, 1]
    mesh = plsc.VectorSubcoreMesh(core_axis_name="c", subcore_axis_name="s")
    f = pl.kernel(
        _sc_body,
        out_type=jax.ShapeDtypeStruct((N_TRIPLES, PAD_W), jnp.float32),
        mesh=mesh,
        compiler_params=pltpu.CompilerParams(
            needs_layout_passes=False, use_tc_tiling_on_sc=False),
        scratch_types=[
            pltpu.VMEM_SHARED((NZ * NZ, IW), jnp.float32),
            pltpu.VMEM_SHARED((NZ * NZ, JW), jnp.float32),
            pltpu.VMEM((N_NODES,), jnp.float32),
            pltpu.VMEM((N_NODES,), jnp.float32),
            pltpu.VMEM((NZ,), jnp.float32),
            pltpu.VMEM((NZ,), jnp.float32),
            pltpu.VMEM((CHUNK,), jnp.int32),
            pltpu.VMEM((CHUNK,), jnp.int32),
            pltpu.VMEM((CHUNK,), jnp.int32),
            pltpu.VMEM((CHUNK,), jnp.int32),
            pltpu.VMEM((CHUNK,), jnp.int32),
            pltpu.VMEM((CHUNK,), jnp.int32),
            pltpu.VMEM((CHUNK,), jnp.int32),
            pltpu.VMEM((CHUNK,), jnp.int32),
            pltpu.VMEM((CHUNK,), jnp.int32),
            pltpu.VMEM((CHUNK,), jnp.int32),
            pltpu.VMEM((CHUNK, TAIL_W), jnp.float32),
            pltpu.VMEM((CHUNK, IW), jnp.float32),
            pltpu.VMEM((CHUNK, JW), jnp.float32),
            pltpu.SemaphoreType.DMA,
        ],
    )
    out = f(triple_idx_i, triple_idx_j, triple_idx_k, idx_j, z, px, py,
            itab, jtab, t20, t21)
    return out[:, :OUT_W]


# speedup vs baseline: 17.3357x; 1.0420x over previous
import jax
import jax.numpy as jnp
import numpy as np
from jax import lax
from jax.experimental import pallas as pl
from jax.experimental.pallas import tpu as pltpu
from jax.experimental.pallas import tpu_sc as plsc

N_NODES = 10000
N_TRIPLES = 640000
CFG_W = 22
OUT_W = 3 * CFG_W + 3
PAD_W = 72
TAIL_W = 8
NZ = 100
IW = 24
JW = 40

_ORB = '1s 2s 2p 3s 3p 4s 3d 4p 5s 4d 5p 6s 4f 5d 6p 7s 5f 6d 7p 6f 7d 7f'.split()
_CAP = dict(s=2, p=6, d=10, f=14)


def _cfg_row(a):
    cnt, last, row = 0, -1, []
    for o in _ORB:
        if cnt < a:
            c = _CAP[o[-1]]
            row.append(c)
            cnt += c
            last += 1
        else:
            row.append(0)
    if cnt > a:
        row[last] -= cnt - a
    return row


_TAB_NP = np.array([_cfg_row(a) for a in range(NZ)], dtype=np.float32)

_ITAB_NP = np.zeros((NZ * NZ, IW), dtype=np.float32)
_ITAB_NP[:, 0:CFG_W] = np.repeat(_TAB_NP, NZ, axis=0)
_ITAB_NP[:, CFG_W:IW] = np.tile(_TAB_NP[:, 0:2], (NZ, 1))

_JTAB_NP = np.zeros((NZ * NZ, JW), dtype=np.float32)
_JTAB_NP[:, 0:CFG_W - 2] = np.repeat(_TAB_NP[:, 2:CFG_W], NZ, axis=0)
_JTAB_NP[:, CFG_W - 2:JW] = np.tile(_TAB_NP[:, 0:CFG_W - 2], (NZ, 1))

_T20_NP = np.ascontiguousarray(_TAB_NP[:, 20])
_T21_NP = np.ascontiguousarray(_TAB_NP[:, 21])

NW = 32
CHUNK = 800
PER_W = N_TRIPLES // NW
N_CH = PER_W // CHUNK
VEC_IT = CHUNK // 16


def _sc_body(ti_hbm, tj_hbm, tk_hbm, idxj_hbm, z_hbm, px_hbm, py_hbm,
             itab_hbm, jtab_hbm, t20_hbm, t21_hbm,
             out_hbm,
             itab_sp, jtab_sp, px_v, py_v, t20_v, t21_v,
             ti_v, tj_v, tk_v, jn_v, kn_v, zi_v, zj_v, zk_v, pidx_v, qidx_v,
             tail_b, ti_b, tj_b, sem):
    cid = lax.axis_index("c")
    sid = lax.axis_index("s")
    wid = sid * 2 + cid
    base = wid * PER_W

    @pl.when(sid == 0)
    def _():
        pltpu.sync_copy(itab_hbm, itab_sp)
        pltpu.sync_copy(jtab_hbm, jtab_sp)

    pltpu.sync_copy(px_hbm, px_v)
    pltpu.sync_copy(py_hbm, py_v)
    pltpu.sync_copy(t20_hbm, t20_v)
    pltpu.sync_copy(t21_hbm, t21_v)
    plsc.subcore_barrier()

    k0 = jnp.full((16,), 0, jnp.int32)
    k1 = jnp.full((16,), 1, jnp.int32)
    k2 = jnp.full((16,), 2, jnp.int32)
    k3 = jnp.full((16,), 3, jnp.int32)
    k4 = jnp.full((16,), 4, jnp.int32)

    def chunk_body(ci, _):
        g0 = base + ci * CHUNK
        a1 = pltpu.async_copy(ti_hbm.at[pl.ds(g0, CHUNK)], ti_v, sem)
        a2 = pltpu.async_copy(tj_hbm.at[pl.ds(g0, CHUNK)], tj_v, sem)
        a3 = pltpu.async_copy(tk_hbm.at[pl.ds(g0, CHUNK)], tk_v, sem)
        a1.wait(); a2.wait(); a3.wait()
        b1 = pltpu.async_copy(idxj_hbm.at[tj_v], jn_v, sem)
        b2 = pltpu.async_copy(idxj_hbm.at[tk_v], kn_v, sem)
        c1 = pltpu.async_copy(z_hbm.at[ti_v], zi_v, sem)
        b1.wait(); b2.wait()
        c2 = pltpu.async_copy(z_hbm.at[jn_v], zj_v, sem)
        c3 = pltpu.async_copy(z_hbm.at[kn_v], zk_v, sem)
        c1.wait(); c2.wait(); c3.wait()

        def pidx_body(vi, _):
            s = pl.ds(vi * 16, 16)
            zj = zj_v[s]
            pidx_v[s] = zj * 100 + zk_v[s]
            qidx_v[s] = zi_v[s] * 100 + zj
            return _

        lax.fori_loop(0, VEC_IT, pidx_body, 0, unroll=4)

        e1 = pltpu.async_copy(itab_sp.at[qidx_v], ti_b, sem)
        e2 = pltpu.async_copy(jtab_sp.at[pidx_v], tj_b, sem)

        def vec_body(vi, _):
            row16 = lax.iota(jnp.int32, 16) + vi * 16
            ti16 = ti_v[pl.ds(vi * 16, 16)]
            jn16 = jn_v[pl.ds(vi * 16, 16)]
            kn16 = kn_v[pl.ds(vi * 16, 16)]
            zk16 = zk_v[pl.ds(vi * 16, 16)]
            pxi = plsc.load_gather(px_v, [ti16])
            pxj = plsc.load_gather(px_v, [jn16])
            pxk = plsc.load_gather(px_v, [kn16])
            pyi = plsc.load_gather(py_v, [ti16])
            pyj = plsc.load_gather(py_v, [jn16])
            pyk = plsc.load_gather(py_v, [kn16])
            t20 = plsc.load_gather(t20_v, [zk16])
            t21 = plsc.load_gather(t21_v, [zk16])
            sx = ((pxj - pxi) + (pxk - pxi)) * 0.5
            y2 = pyj - pyi
            y3 = pyk - pyi
            plsc.store_scatter(tail_b, [row16, k0], t20)
            plsc.store_scatter(tail_b, [row16, k1], t21)
            plsc.store_scatter(tail_b, [row16, k2], sx)
            plsc.store_scatter(tail_b, [row16, k3], y2)
            plsc.store_scatter(tail_b, [row16, k4], y3)
            return _

        lax.fori_loop(0, VEC_IT, vec_body, 0, unroll=2)
        e1.wait(); e2.wait()
        h0 = pltpu.async_copy(
            tail_b, out_hbm.at[pl.ds(g0, CHUNK), pl.ds(JW + IW, TAIL_W)], sem)
        h1 = pltpu.async_copy(
            ti_b, out_hbm.at[pl.ds(g0, CHUNK), pl.ds(0, IW)], sem)
        h2 = pltpu.async_copy(
            tj_b, out_hbm.at[pl.ds(g0, CHUNK), pl.ds(IW, JW)], sem)
        h0.wait(); h1.wait(); h2.wait()
        return _

    lax.fori_loop(0, N_CH, chunk_body, 0)


@jax.jit
def kernel(triple_idx_i, triple_idx_j, triple_idx_k, idx_i, idx_j, z, positions):
    del idx_i
    itab = jnp.asarray(_ITAB_NP)
    jtab = jnp.asarray(_JTAB_NP)
    t20 = jnp.asarray(_T20_NP)
    t21 = jnp.asarray(_T21_NP)
    px = positions[:, 0]
    py = positions[:, 1]
    mesh = plsc.VectorSubcoreMesh(core_axis_name="c", subcore_axis_name="s")
    f = pl.kernel(
        _sc_body,
        out_type=jax.ShapeDtypeStruct((N_TRIPLES, PAD_W), jnp.float32),
        mesh=mesh,
        compiler_params=pltpu.CompilerParams(
            needs_layout_passes=False, use_tc_tiling_on_sc=False),
        scratch_types=[
            pltpu.VMEM_SHARED((NZ * NZ, IW), jnp.float32),
            pltpu.VMEM_SHARED((NZ * NZ, JW), jnp.float32),
            pltpu.VMEM((N_NODES,), jnp.float32),
            pltpu.VMEM((N_NODES,), jnp.float32),
            pltpu.VMEM((NZ,), jnp.float32),
            pltpu.VMEM((NZ,), jnp.float32),
            pltpu.VMEM((CHUNK,), jnp.int32),
            pltpu.VMEM((CHUNK,), jnp.int32),
            pltpu.VMEM((CHUNK,), jnp.int32),
            pltpu.VMEM((CHUNK,), jnp.int32),
            pltpu.VMEM((CHUNK,), jnp.int32),
            pltpu.VMEM((CHUNK,), jnp.int32),
            pltpu.VMEM((CHUNK,), jnp.int32),
            pltpu.VMEM((CHUNK,), jnp.int32),
            pltpu.VMEM((CHUNK,), jnp.int32),
            pltpu.VMEM((CHUNK,), jnp.int32),
            pltpu.VMEM((CHUNK, TAIL_W), jnp.float32),
            pltpu.VMEM((CHUNK, IW), jnp.float32),
            pltpu.VMEM((CHUNK, JW), jnp.float32),
            pltpu.SemaphoreType.DMA,
        ],
    )
    out = f(triple_idx_i, triple_idx_j, triple_idx_k, idx_j, z, px, py,
            itab, jtab, t20, t21)
    return out[:, :OUT_W]
